# baseline (speedup 1.0000x reference)
import functools

import jax
import jax.numpy as jnp
from jax import lax
from jax.experimental import pallas as pl
from jax.experimental.pallas import tpu as pltpu
from jax.experimental.pallas import tpu_sc as plsc

_N, _D, _E = 10000, 128, 320000
_NC, _NS = 2, 16
_NW = _NC * _NS
_EPT = _E // _NW
_K = 80
_NCH = _EPT // _K
_NP = 10240
_RPT = _NP // _NS

_mesh = plsc.VectorSubcoreMesh(core_axis_name="c", subcore_axis_name="s",
                               num_cores=_NC, num_subcores=_NS)


@functools.partial(
    pl.kernel,
    out_type=jax.ShapeDtypeStruct((_NC, _NP), jnp.float32),
    mesh=_mesh,
    scratch_types=[
        pltpu.VMEM((_NCH, _K), jnp.int32),
        pltpu.VMEM((_NCH, _K), jnp.float32),
        pltpu.VMEM((_RPT,), jnp.float32),
        pltpu.VMEM_SHARED((_NP,), jnp.float32),
    ],
    compiler_params=pltpu.CompilerParams(needs_layout_passes=False),
)
def _sc_deg(cols_hbm, ew_hbm, out_hbm, col_v, ew_v, zbuf, acc):
    c = lax.axis_index("c")
    s = lax.axis_index("s")
    w = c * _NS + s

    def zero_body(i, _):
        zbuf[pl.ds(i * 16, 16)] = jnp.zeros((16,), jnp.float32)
        return 0

    lax.fori_loop(0, _RPT // 16, zero_body, 0)
    pltpu.sync_copy(zbuf, acc.at[pl.ds(s * _RPT, _RPT)])
    plsc.subcore_barrier()

    pltpu.sync_copy(cols_hbm.at[w], col_v)
    pltpu.sync_copy(ew_hbm.at[w], ew_v)

    def chunk_body(j, _):
        pltpu.sync_copy(ew_v.at[j], acc.at[col_v.at[j]], add=True)
        return 0

    lax.fori_loop(0, _NCH, chunk_body, 0)
    plsc.subcore_barrier()
    pltpu.sync_copy(acc.at[pl.ds(s * _RPT, _RPT)],
                    out_hbm.at[c, pl.ds(s * _RPT, _RPT)])


_DH = _D // 2
_EPS = _E // _NS
_NCHM = _EPS // _K


@functools.partial(
    pl.kernel,
    out_type=jax.ShapeDtypeStruct((_NC, _NP, _DH), jnp.float32),
    mesh=_mesh,
    scratch_types=[
        pltpu.VMEM((_EPS,), jnp.int32),
        pltpu.VMEM((_NCHM, _K), jnp.int32),
        pltpu.VMEM((_EPS,), jnp.float32),
        pltpu.VMEM((2, _K, _DH // 2), jnp.int32),
        pltpu.VMEM((2, _K, _DH), jnp.float32),
        pltpu.VMEM_SHARED((_NP, _DH), jnp.float32),
        pltpu.SemaphoreType.DMA,
        pltpu.SemaphoreType.DMA,
        pltpu.SemaphoreType.DMA,
        pltpu.SemaphoreType.DMA,
    ],
    compiler_params=pltpu.CompilerParams(needs_layout_passes=False,
                                         use_tc_tiling_on_sc=False),
)
def _sc_msg(xwd_hbm, rows_hbm, cols_hbm, ew_hbm, out_hbm,
            row_v, col_v, ew_v, gbuf, fbuf, acc, sem0, sem1, ssem0, ssem1):
    c = lax.axis_index("c")
    s = lax.axis_index("s")

    def zero_body(r, _):
        for f in range(_DH // 16):
            fbuf[0, r, pl.ds(f * 16, 16)] = jnp.zeros((16,), jnp.float32)
        return 0

    lax.fori_loop(0, _K, zero_body, 0)
    for i in range(_RPT // _K):
        pltpu.sync_copy(fbuf.at[0], acc.at[pl.ds(s * _RPT + i * _K, _K)])
    plsc.subcore_barrier()

    pltpu.sync_copy(rows_hbm.at[s], row_v)
    pltpu.sync_copy(cols_hbm.at[s], col_v)
    pltpu.sync_copy(ew_hbm.at[s], ew_v)

    src = xwd_hbm.at[c]
    _HI = jnp.uint32(0xFFFF0000)

    def gather_idx(j):
        return row_v.at[pl.ds(pl.multiple_of(j * _K, 8), _K)]

    pltpu.async_copy(src.at[gather_idx(0)], gbuf.at[0], sem0)

    def pair_body(p, _):
        j0 = p * 2

        def process(j, slot, my_sem, other_sem, my_ssem, has_next, has_prev2):
            pltpu.make_async_copy(src.at[gather_idx(j)],
                                  gbuf.at[slot], my_sem).wait()

            @pl.when(has_next)
            def _():
                pltpu.async_copy(src.at[gather_idx(j + 1)],
                                 gbuf.at[1 - slot], other_sem)

            @pl.when(has_prev2)
            def _():
                pltpu.make_async_copy(fbuf.at[slot],
                                      acc.at[col_v.at[j - 2]],
                                      my_ssem).wait()

            def scale_body(g, _):
                base = j * _K + g * 16
                for i in range(16):
                    splat = plsc.load_gather(
                        ew_v, [jnp.broadcast_to(base + i, (16,))])
                    r = g * 16 + i
                    for f in range(_DH // 32):
                        v = plsc.bitcast(gbuf[slot, r, pl.ds(f * 16, 16)],
                                         jnp.uint32)
                        lo = plsc.bitcast(v << 16, jnp.float32)
                        hi = plsc.bitcast(v & _HI, jnp.float32)
                        fbuf[slot, r, pl.ds(32 * f, 16)] = lo * splat
                        fbuf[slot, r, pl.ds(32 * f + 16, 16)] = hi * splat
                return 0

            lax.fori_loop(0, _K // 16, scale_body, 0)
            pltpu.async_copy(fbuf.at[slot], acc.at[col_v.at[j]], my_ssem,
                             add=True)

        process(j0, 0, sem0, sem1, ssem0, j0 + 1 < _NCHM, j0 > 0)
        process(j0 + 1, 1, sem1, sem0, ssem1, j0 + 2 < _NCHM, j0 > 0)
        return 0

    lax.fori_loop(0, _NCHM // 2, pair_body, 0)
    pltpu.make_async_copy(fbuf.at[0], acc.at[col_v.at[_NCHM - 2]],
                          ssem0).wait()
    pltpu.make_async_copy(fbuf.at[1], acc.at[col_v.at[_NCHM - 1]],
                          ssem1).wait()

    plsc.subcore_barrier()
    pltpu.sync_copy(acc.at[pl.ds(s * _RPT, _RPT)],
                    out_hbm.at[c, pl.ds(s * _RPT, _RPT)])


def _pack_pair(xwd, out_ref):
    for h in range(2):
        xh = xwd[:, h * _DH:(h + 1) * _DH]
        cols = []
        for f in range(2):
            lo = xh[:, 32 * f:32 * f + 16].astype(jnp.bfloat16)
            hi = xh[:, 32 * f + 16:32 * f + 32].astype(jnp.bfloat16)
            lo_b = lax.bitcast_convert_type(lo, jnp.uint16).astype(jnp.uint32)
            hi_b = lax.bitcast_convert_type(hi, jnp.uint16).astype(jnp.uint32)
            cols.append(
                lax.bitcast_convert_type(lo_b | (hi_b << 16), jnp.int32))
        out_ref[h] = jnp.concatenate(cols, axis=1)


def _tc_mm1_kernel(x_ref, w_ref, degp_ref, xw_ref, xwd_ref, dis_ref, inv_ref):
    deg = degp_ref[0] + degp_ref[1] + 1.0
    dis = jnp.where(deg > 0, lax.rsqrt(jnp.maximum(deg, 1e-12)), 0.0)
    dis_ref[...] = dis
    inv_ref[...] = dis * dis
    xw = lax.dot_general(x_ref[...], w_ref[...], (((1,), (1,)), ((), ())),
                         preferred_element_type=jnp.float32)
    xw_ref[...] = xw
    _pack_pair(xw * dis, xwd_ref)


def _tc_mid_kernel(p_ref, xw_ref, dis_ref, inv_ref, b_ref, w_ref,
                   xw2_ref, xwd2_ref):
    p = jnp.concatenate([p_ref[0], p_ref[1]], axis=1)
    h = jnp.maximum(p * dis_ref[...] + xw_ref[...] * inv_ref[...]
                    + b_ref[...], 0.0)
    xw2 = lax.dot_general(h, w_ref[...], (((1,), (1,)), ((), ())),
                          preferred_element_type=jnp.float32)
    xw2_ref[...] = xw2
    _pack_pair(xw2 * dis_ref[...], xwd2_ref)


def _tc_fin_kernel(p_ref, xw_ref, dis_ref, inv_ref, b_ref,
                   cw1_ref, cb1_ref, cw2_ref, cb2_ref, out_ref):
    p = jnp.concatenate([p_ref[0], p_ref[1]], axis=1)
    h = jnp.maximum(p * dis_ref[...] + xw_ref[...] * inv_ref[...]
                    + b_ref[...], 0.0)
    t0 = jnp.maximum(cw1_ref[0, 0] * h[:, 0:126] + cw1_ref[0, 1] * h[:, 1:127]
                     + cw1_ref[0, 2] * h[:, 2:128] + cb1_ref[0], 0.0)
    t1 = jnp.maximum(cw1_ref[1, 0] * h[:, 0:126] + cw1_ref[1, 1] * h[:, 1:127]
                     + cw1_ref[1, 2] * h[:, 2:128] + cb1_ref[1], 0.0)
    o = (cw2_ref[0, 0] * t0[:, 0:124] + cw2_ref[0, 1] * t0[:, 1:125]
         + cw2_ref[0, 2] * t0[:, 2:126]
         + cw2_ref[1, 0] * t1[:, 0:124] + cw2_ref[1, 1] * t1[:, 1:125]
         + cw2_ref[1, 2] * t1[:, 2:126] + cb2_ref[0])
    out_ref[...] = jnp.maximum(o, 0.0)


_R = 1000


def _row_spec(shape_tail):
    return pl.BlockSpec((_R,) + shape_tail, lambda i: (i,) + (0,) * len(shape_tail))


_pair_spec = pl.BlockSpec((2, _R, _DH), lambda i: (0, i, 0))
_pair_shape = jax.ShapeDtypeStruct((2, _N, _DH), jnp.float32)
_packed_spec = pl.BlockSpec((2, _R, _DH // 2), lambda i: (0, i, 0))
_packed_shape = jax.ShapeDtypeStruct((2, _N, _DH // 2), jnp.int32)

_tc_mm1 = pl.pallas_call(
    _tc_mm1_kernel,
    grid=(_N // _R,),
    in_specs=[
        _row_spec((_D,)),
        pl.BlockSpec((_D, _D), lambda i: (0, 0)),
        pl.BlockSpec((2, _R, 1), lambda i: (0, i, 0)),
    ],
    out_specs=(_row_spec((_D,)), _packed_spec, _row_spec((1,)),
               _row_spec((1,))),
    out_shape=(jax.ShapeDtypeStruct((_N, _D), jnp.float32), _packed_shape,
               jax.ShapeDtypeStruct((_N, 1), jnp.float32),
               jax.ShapeDtypeStruct((_N, 1), jnp.float32)),
)

_tc_mid = pl.pallas_call(
    _tc_mid_kernel,
    grid=(_N // _R,),
    in_specs=[
        _pair_spec,
        _row_spec((_D,)),
        _row_spec((1,)),
        _row_spec((1,)),
        pl.BlockSpec((1, _D), lambda i: (0, 0)),
        pl.BlockSpec((_D, _D), lambda i: (0, 0)),
    ],
    out_specs=(_row_spec((_D,)), _packed_spec),
    out_shape=(jax.ShapeDtypeStruct((_N, _D), jnp.float32), _packed_shape),
)

_tc_fin = pl.pallas_call(
    _tc_fin_kernel,
    grid=(_N // _R,),
    in_specs=[
        _pair_spec,
        _row_spec((_D,)),
        _row_spec((1,)),
        _row_spec((1,)),
        pl.BlockSpec((1, _D), lambda i: (0, 0)),
        pl.BlockSpec(memory_space=pltpu.SMEM),
        pl.BlockSpec(memory_space=pltpu.SMEM),
        pl.BlockSpec(memory_space=pltpu.SMEM),
        pl.BlockSpec(memory_space=pltpu.SMEM),
    ],
    out_specs=_row_spec((124,)),
    out_shape=jax.ShapeDtypeStruct((_N, 124), jnp.float32),
)


def kernel(X, edge_index, edge_weight, W1, b1, W2, b2, cw1, cb1, cw2, cb2):
    cols_deg = edge_index[1].reshape(_NW, _NCH, _K)
    ew_deg = edge_weight.reshape(_NW, _NCH, _K)
    rows_m = edge_index[0].reshape(_NS, _EPS)
    cols_m = edge_index[1].reshape(_NS, _NCHM, _K)
    ew_m = edge_weight.reshape(_NS, _EPS)

    degp = _sc_deg(cols_deg, ew_deg)
    xw1, xwd1, dis_col, inv_col = _tc_mm1(X, W1, degp.reshape(_NC, _NP, 1))
    p1 = _sc_msg(xwd1, rows_m, cols_m, ew_m)[:, :_N]
    xw2, xwd2 = _tc_mid(p1, xw1, dis_col, inv_col, b1.reshape(1, _D), W2)
    p2 = _sc_msg(xwd2, rows_m, cols_m, ew_m)[:, :_N]
    return _tc_fin(p2, xw2, dis_col, inv_col, b2.reshape(1, _D),
                   cw1.reshape(2, 3), cb1, cw2.reshape(2, 3), cb2)


# speedup vs baseline: 1.3780x; 1.3780x over previous
import functools

import jax
import jax.numpy as jnp
from jax import lax
from jax.experimental import pallas as pl
from jax.experimental.pallas import tpu as pltpu
from jax.experimental.pallas import tpu_sc as plsc

_N, _D, _E = 10000, 128, 320000
_NC, _NS = 2, 16
_NW = _NC * _NS
_EPT = _E // _NW
_K = 80
_NCH = _EPT // _K
_NP = 10240
_RPT = _NP // _NS

_mesh = plsc.VectorSubcoreMesh(core_axis_name="c", subcore_axis_name="s",
                               num_cores=_NC, num_subcores=_NS)


@functools.partial(
    pl.kernel,
    out_type=jax.ShapeDtypeStruct((_NC, _NP), jnp.float32),
    mesh=_mesh,
    scratch_types=[
        pltpu.VMEM((_NCH, _K), jnp.int32),
        pltpu.VMEM((_NCH, _K), jnp.float32),
        pltpu.VMEM((_RPT,), jnp.float32),
        pltpu.VMEM_SHARED((_NP,), jnp.float32),
    ],
    compiler_params=pltpu.CompilerParams(needs_layout_passes=False),
)
def _sc_deg(cols_hbm, ew_hbm, out_hbm, col_v, ew_v, zbuf, acc):
    c = lax.axis_index("c")
    s = lax.axis_index("s")
    w = c * _NS + s

    def zero_body(i, _):
        zbuf[pl.ds(i * 16, 16)] = jnp.zeros((16,), jnp.float32)
        return 0

    lax.fori_loop(0, _RPT // 16, zero_body, 0)
    pltpu.sync_copy(zbuf, acc.at[pl.ds(s * _RPT, _RPT)])
    plsc.subcore_barrier()

    pltpu.sync_copy(cols_hbm.at[w], col_v)
    pltpu.sync_copy(ew_hbm.at[w], ew_v)

    def chunk_body(j, _):
        pltpu.sync_copy(ew_v.at[j], acc.at[col_v.at[j]], add=True)
        return 0

    lax.fori_loop(0, _NCH, chunk_body, 0)
    plsc.subcore_barrier()
    pltpu.sync_copy(acc.at[pl.ds(s * _RPT, _RPT)],
                    out_hbm.at[c, pl.ds(s * _RPT, _RPT)])


_EPW = _E // _NW
_NCHM = _EPW // _K


@functools.partial(
    pl.kernel,
    out_type=jax.ShapeDtypeStruct((_NC, _NP, _D), jnp.float32),
    mesh=_mesh,
    scratch_types=[
        pltpu.VMEM((_EPW,), jnp.int32),
        pltpu.VMEM((_EPW,), jnp.int32),
        pltpu.VMEM((_EPW // 2,), jnp.int32),
        pltpu.VMEM((2, _K, _D), jnp.float32),
        pltpu.VMEM_SHARED((_NP, _D), jnp.float32),
        pltpu.SemaphoreType.DMA,
        pltpu.SemaphoreType.DMA,
        pltpu.SemaphoreType.DMA,
        pltpu.SemaphoreType.DMA,
    ],
    compiler_params=pltpu.CompilerParams(needs_layout_passes=False,
                                         use_tc_tiling_on_sc=False),
)
def _sc_msg(xwd_hbm, rows_hbm, cols_hbm, ewp_hbm, out_hbm,
            row_v, col_v, ewp_v, fbuf, acc, sem0, sem1, ssem0, ssem1):
    c = lax.axis_index("c")
    s = lax.axis_index("s")
    w = c * _NS + s

    def zero_body(r, _):
        for f in range(_D // 16):
            fbuf[0, r, pl.ds(f * 16, 16)] = jnp.zeros((16,), jnp.float32)
        return 0

    lax.fori_loop(0, _K, zero_body, 0)
    for i in range(_RPT // _K):
        pltpu.sync_copy(fbuf.at[0], acc.at[pl.ds(s * _RPT + i * _K, _K)])
    plsc.subcore_barrier()

    pltpu.sync_copy(rows_hbm.at[w], row_v)
    pltpu.sync_copy(cols_hbm.at[w], col_v)
    pltpu.sync_copy(ewp_hbm.at[w], ewp_v)

    _HI = jnp.uint32(0xFFFF0000)

    def gather_idx(j):
        return row_v.at[pl.ds(pl.multiple_of(j * _K, 8), _K)]

    def scatter_idx(j):
        return col_v.at[pl.ds(pl.multiple_of(j * _K, 8), _K)]

    pltpu.async_copy(xwd_hbm.at[gather_idx(0)], fbuf.at[0], sem0)

    def process(j, slot, my_sem, other_sem, my_ssem, has_next, has_prev2):
        pltpu.make_async_copy(xwd_hbm.at[gather_idx(j)],
                              fbuf.at[slot], my_sem).wait()

        @pl.when(has_prev2)
        def _():
            pltpu.make_async_copy(fbuf.at[slot],
                                  acc.at[scatter_idx(j - 2)],
                                  my_ssem).wait()

        @pl.when(has_next)
        def _():
            pltpu.async_copy(xwd_hbm.at[gather_idx(j + 1)],
                             fbuf.at[1 - slot], other_sem)

        def scale_body(g, _):
            half = j * (_K // 2) + g * 8
            for i in range(16):
                v = plsc.bitcast(
                    plsc.load_gather(
                        ewp_v, [jnp.broadcast_to(half + i // 2, (16,))]),
                    jnp.uint32)
                if i % 2 == 0:
                    splat = plsc.bitcast(v << 16, jnp.float32)
                else:
                    splat = plsc.bitcast(v & _HI, jnp.float32)
                r = g * 16 + i
                for f in range(_D // 16):
                    sl = pl.ds(f * 16, 16)
                    fbuf[slot, r, sl] = fbuf[slot, r, sl] * splat
            return 0

        lax.fori_loop(0, _K // 16, scale_body, 0)
        pltpu.async_copy(fbuf.at[slot], acc.at[scatter_idx(j)], my_ssem,
                         add=True)

    def pair_body(p, _):
        j0 = p * 2
        process(j0, 0, sem0, sem1, ssem0, j0 + 1 < _NCHM, j0 > 0)
        process(j0 + 1, 1, sem1, sem0, ssem1, j0 + 2 < _NCHM, j0 > 0)
        return 0

    lax.fori_loop(0, _NCHM // 2, pair_body, 0)
    process(_NCHM - 1, 0, sem0, sem1, ssem0, False, True)
    pltpu.make_async_copy(fbuf.at[1], acc.at[scatter_idx(_NCHM - 2)],
                          ssem1).wait()
    pltpu.make_async_copy(fbuf.at[0], acc.at[scatter_idx(_NCHM - 1)],
                          ssem0).wait()

    plsc.subcore_barrier()
    pltpu.sync_copy(acc.at[pl.ds(s * _RPT, _RPT)],
                    out_hbm.at[c, pl.ds(s * _RPT, _RPT)])


def _tc_mm1_kernel(x_ref, w_ref, degp_ref, xw_ref, xwd_ref, dis_ref, inv_ref):
    deg = degp_ref[0] + degp_ref[1] + 1.0
    dis = jnp.where(deg > 0, lax.rsqrt(jnp.maximum(deg, 1e-12)), 0.0)
    dis_ref[...] = dis
    inv_ref[...] = dis * dis
    xw = lax.dot_general(x_ref[...], w_ref[...], (((1,), (1,)), ((), ())),
                         preferred_element_type=jnp.float32)
    xw_ref[...] = xw
    xwd_ref[...] = xw * dis


def _tc_mid_kernel(p_ref, xw_ref, dis_ref, inv_ref, b_ref, w_ref,
                   xw2_ref, xwd2_ref):
    p = p_ref[0] + p_ref[1]
    h = jnp.maximum(p * dis_ref[...] + xw_ref[...] * inv_ref[...]
                    + b_ref[...], 0.0)
    xw2 = lax.dot_general(h, w_ref[...], (((1,), (1,)), ((), ())),
                          preferred_element_type=jnp.float32)
    xw2_ref[...] = xw2
    xwd2_ref[...] = xw2 * dis_ref[...]


def _tc_fin_kernel(p_ref, xw_ref, dis_ref, inv_ref, b_ref,
                   cw1_ref, cb1_ref, cw2_ref, cb2_ref, out_ref):
    p = p_ref[0] + p_ref[1]
    h = jnp.maximum(p * dis_ref[...] + xw_ref[...] * inv_ref[...]
                    + b_ref[...], 0.0)
    t0 = jnp.maximum(cw1_ref[0, 0] * h[:, 0:126] + cw1_ref[0, 1] * h[:, 1:127]
                     + cw1_ref[0, 2] * h[:, 2:128] + cb1_ref[0], 0.0)
    t1 = jnp.maximum(cw1_ref[1, 0] * h[:, 0:126] + cw1_ref[1, 1] * h[:, 1:127]
                     + cw1_ref[1, 2] * h[:, 2:128] + cb1_ref[1], 0.0)
    o = (cw2_ref[0, 0] * t0[:, 0:124] + cw2_ref[0, 1] * t0[:, 1:125]
         + cw2_ref[0, 2] * t0[:, 2:126]
         + cw2_ref[1, 0] * t1[:, 0:124] + cw2_ref[1, 1] * t1[:, 1:125]
         + cw2_ref[1, 2] * t1[:, 2:126] + cb2_ref[0])
    out_ref[...] = jnp.maximum(o, 0.0)


_R = 1000


def _row_spec(shape_tail):
    return pl.BlockSpec((_R,) + shape_tail, lambda i: (i,) + (0,) * len(shape_tail))


_pair_spec = pl.BlockSpec((2, _R, _D), lambda i: (0, i, 0))

_tc_mm1 = pl.pallas_call(
    _tc_mm1_kernel,
    grid=(_N // _R,),
    in_specs=[
        _row_spec((_D,)),
        pl.BlockSpec((_D, _D), lambda i: (0, 0)),
        pl.BlockSpec((2, _R, 1), lambda i: (0, i, 0)),
    ],
    out_specs=(_row_spec((_D,)), _row_spec((_D,)), _row_spec((1,)),
               _row_spec((1,))),
    out_shape=(jax.ShapeDtypeStruct((_N, _D), jnp.float32),
               jax.ShapeDtypeStruct((_N, _D), jnp.float32),
               jax.ShapeDtypeStruct((_N, 1), jnp.float32),
               jax.ShapeDtypeStruct((_N, 1), jnp.float32)),
)

_tc_mid = pl.pallas_call(
    _tc_mid_kernel,
    grid=(_N // _R,),
    in_specs=[
        _pair_spec,
        _row_spec((_D,)),
        _row_spec((1,)),
        _row_spec((1,)),
        pl.BlockSpec((1, _D), lambda i: (0, 0)),
        pl.BlockSpec((_D, _D), lambda i: (0, 0)),
    ],
    out_specs=(_row_spec((_D,)), _row_spec((_D,))),
    out_shape=(jax.ShapeDtypeStruct((_N, _D), jnp.float32),
               jax.ShapeDtypeStruct((_N, _D), jnp.float32)),
)

_tc_fin = pl.pallas_call(
    _tc_fin_kernel,
    grid=(_N // _R,),
    in_specs=[
        _pair_spec,
        _row_spec((_D,)),
        _row_spec((1,)),
        _row_spec((1,)),
        pl.BlockSpec((1, _D), lambda i: (0, 0)),
        pl.BlockSpec(memory_space=pltpu.SMEM),
        pl.BlockSpec(memory_space=pltpu.SMEM),
        pl.BlockSpec(memory_space=pltpu.SMEM),
        pl.BlockSpec(memory_space=pltpu.SMEM),
    ],
    out_specs=_row_spec((124,)),
    out_shape=jax.ShapeDtypeStruct((_N, 124), jnp.float32),
)


def kernel(X, edge_index, edge_weight, W1, b1, W2, b2, cw1, cb1, cw2, cb2):
    cols_deg = edge_index[1].reshape(_NW, _NCH, _K)
    ew_deg = edge_weight.reshape(_NW, _NCH, _K)
    rows_m = edge_index[0].reshape(_NW, _EPW)
    cols_m = edge_index[1].reshape(_NW, _EPW)
    ewb = lax.bitcast_convert_type(edge_weight.astype(jnp.bfloat16),
                                   jnp.uint16).astype(jnp.uint32)
    ewp = lax.bitcast_convert_type(ewb[0::2] | (ewb[1::2] << 16),
                                   jnp.int32).reshape(_NW, _EPW // 2)

    degp = _sc_deg(cols_deg, ew_deg)
    xw1, xwd1, dis_col, inv_col = _tc_mm1(X, W1, degp.reshape(_NC, _NP, 1))
    p1 = _sc_msg(xwd1, rows_m, cols_m, ewp)[:, :_N]
    xw2, xwd2 = _tc_mid(p1, xw1, dis_col, inv_col, b1.reshape(1, _D), W2)
    p2 = _sc_msg(xwd2, rows_m, cols_m, ewp)[:, :_N]
    return _tc_fin(p2, xw2, dis_col, inv_col, b2.reshape(1, _D),
                   cw1.reshape(2, 3), cb1, cw2.reshape(2, 3), cb2)
